# baseline (speedup 1.0000x reference)
import jax
import jax.numpy as jnp
from jax import lax
from jax.experimental import pallas as pl
from jax.experimental.pallas import tpu as pltpu

_ORDER = (0, 1, 3, 2)


def _fused_kernel(
    emb_ref,
    wif_ref,
    wib_ref,
    whf_ref,
    whb_ref,
    bf_ref,
    bb_ref,
    cwt_ref,
    convb_ref,
    fcw_ref,
    fcb_ref,
    out_ref,
    wih_s,
    whh_s,
    xg_ref,
    hall_ref,
):
    T, BB, E = emb_ref.shape
    H4, H = whf_ref.shape
    H2 = 2 * H
    G8 = 8 * H
    Cp = convb_ref.shape[1]
    CP3 = 3 * Cp
    NC = fcw_ref.shape[0]
    NCp = out_ref.shape[1]
    TC = 8 if T % 8 == 0 else 1

    def pack_T(wf_ref2, wb_ref2, ncols):
        z = jnp.zeros((H, ncols), jnp.float32)
        blocks = []
        for g in _ORDER:
            blocks.append(jnp.concatenate(
                [wf_ref2[g * H:(g + 1) * H, :], z], axis=1))
            blocks.append(jnp.concatenate(
                [z, wb_ref2[g * H:(g + 1) * H, :]], axis=1))
        return jnp.concatenate(blocks, axis=0)

    wih_s[...] = jnp.transpose(pack_T(wif_ref, wib_ref, E)).astype(jnp.bfloat16)
    whh_s[...] = jnp.transpose(pack_T(whf_ref, whb_ref, H)).astype(jnp.bfloat16)
    bias = jnp.concatenate(
        [jnp.concatenate([bf_ref[:, g * H:(g + 1) * H],
                          bb_ref[:, g * H:(g + 1) * H]], axis=1)
         for g in _ORDER], axis=1)

    wih = wih_s[...]
    whh = whh_s[...]
    hall_ref[0] = jnp.zeros((BB, H2), jnp.bfloat16)
    hall_ref[T + 1] = jnp.zeros((BB, H2), jnp.bfloat16)

    for c in range(T // TC):
        fwd = emb_ref[pl.ds(c * TC, TC)]
        bwd = jnp.stack(
            [emb_ref[T - 1 - (c * TC + k)] for k in range(TC)], axis=0)
        comb = jnp.concatenate([fwd, bwd], axis=-1).astype(
            jnp.bfloat16).reshape(TC * BB, 2 * E)
        xg = jnp.dot(comb, wih, preferred_element_type=jnp.float32) + bias
        xg_ref[pl.ds(c * TC, TC)] = xg.reshape(TC, BB, G8)

    BH = BB // 2
    h0 = jnp.zeros((BH, H2), jnp.bfloat16)
    c0 = jnp.zeros((BH, H2), jnp.float32)

    def substep(t, h, cc, lo):
        gates = (jnp.dot(h, whh, preferred_element_type=jnp.float32)
                 + xg_ref[t, pl.ds(lo, BH)])
        s = 0.5 * jnp.tanh(0.5 * gates[:, :3 * H2]) + 0.5
        g = jnp.tanh(gates[:, 3 * H2:])
        c_new = s[:, H2:2 * H2] * cc + s[:, :H2] * g
        h_new = s[:, 2 * H2:3 * H2] * jnp.tanh(c_new)
        hb = h_new.astype(jnp.bfloat16)
        hall_ref[t + 1, pl.ds(lo, BH), :H] = hb[:, :H]
        hall_ref[T - t, pl.ds(lo, BH), H:] = hb[:, H:]
        return hb, c_new

    def step(t, carry):
        hA, cA, hB, cB = carry
        hA, cA = substep(t, hA, cA, 0)
        hB, cB = substep(t, hB, cB, BH)
        return hA, cA, hB, cB

    lax.fori_loop(0, T, step, (h0, c0, h0, c0), unroll=8)

    def tap(k, rows):
        return cwt_ref[rows, pl.ds(k * Cp, Cp)]
    w1 = jnp.concatenate(
        [tap(0, slice(0, H)), tap(0, slice(H, 2 * H)),
         tap(1, slice(0, H)), tap(1, slice(H, 2 * H)),
         tap(2, slice(0, H)), tap(2, slice(H, 2 * H))], axis=0)
    zc = jnp.zeros((H2, Cp), jnp.float32)
    wacc2 = jnp.concatenate(
        [jnp.concatenate([w1, zc], axis=0),
         jnp.concatenate([zc, w1], axis=0)], axis=1).astype(jnp.bfloat16)

    m2 = jnp.full((BB, 2 * Cp), -jnp.inf, dtype=jnp.float32)
    for p in range(T // 2):
        win = jnp.concatenate(
            [hall_ref[2 * p], hall_ref[2 * p + 1],
             hall_ref[2 * p + 2], hall_ref[2 * p + 3]], axis=1)
        m2 = jnp.maximum(m2, jnp.dot(win, wacc2,
                                     preferred_element_type=jnp.float32))
    m = jnp.maximum(m2[:, :Cp], m2[:, Cp:])
    if T % 2:
        t = T - 1
        win = jnp.concatenate(
            [hall_ref[t], hall_ref[t + 1], hall_ref[t + 2]], axis=1)
        m = jnp.maximum(m, jnp.dot(
            win, wacc2[:3 * H2, :Cp], preferred_element_type=jnp.float32))
    pooled = jnp.maximum(m + convb_ref[...], 0.0)

    logits = lax.dot_general(
        pooled.astype(jnp.bfloat16), fcw_ref[...].astype(jnp.bfloat16),
        (((1,), (1,)), ((), ())),
        preferred_element_type=jnp.float32) + fcb_ref[...]
    out_ref[...] = jnp.concatenate(
        [logits, jnp.zeros((BB, NCp - NC), jnp.float32)], axis=1)


def kernel(x_tokens, embedding, w_ih_f, w_hh_f, b_ih_f, b_hh_f,
           w_ih_b, w_hh_b, b_ih_b, b_hh_b, conv_w, conv_b, fc_w, fc_b):
    B, T = x_tokens.shape
    E = embedding.shape[1]
    H = w_hh_f.shape[1]
    C = conv_w.shape[0]
    NC = fc_b.shape[0]

    BB = 256
    Bp = ((B + BB - 1) // BB) * BB
    Cp = 128
    NCp = 128

    cwt = jnp.transpose(conv_w, (1, 2, 0))
    cwt = jnp.pad(cwt, ((0, 0), (0, 0), (0, Cp - C))).reshape(2 * H, 3 * Cp)
    convb = jnp.zeros((1, Cp), jnp.float32).at[0, :C].set(conv_b)

    if Bp != B:
        x_tokens = jnp.pad(x_tokens, ((0, Bp - B), (0, 0)))

    def half(tok_blk):
        emb = embedding[tok_blk.T]
        return pl.pallas_call(
            _fused_kernel,
            out_shape=jax.ShapeDtypeStruct((BB, NCp), jnp.float32),
            grid_spec=pltpu.PrefetchScalarGridSpec(
                num_scalar_prefetch=0,
                grid=(1,),
                in_specs=[
                    pl.BlockSpec((T, BB, E), lambda i: (0, 0, 0)),
                    pl.BlockSpec((4 * H, E), lambda i: (0, 0)),
                    pl.BlockSpec((4 * H, E), lambda i: (0, 0)),
                    pl.BlockSpec((4 * H, H), lambda i: (0, 0)),
                    pl.BlockSpec((4 * H, H), lambda i: (0, 0)),
                    pl.BlockSpec((1, 4 * H), lambda i: (0, 0)),
                    pl.BlockSpec((1, 4 * H), lambda i: (0, 0)),
                    pl.BlockSpec((2 * H, 3 * Cp), lambda i: (0, 0)),
                    pl.BlockSpec((1, Cp), lambda i: (0, 0)),
                    pl.BlockSpec((NC, C), lambda i: (0, 0)),
                    pl.BlockSpec((1, NC), lambda i: (0, 0)),
                ],
                out_specs=pl.BlockSpec((BB, NCp), lambda i: (0, 0)),
                scratch_shapes=[
                    pltpu.VMEM((2 * E, 8 * H), jnp.bfloat16),
                    pltpu.VMEM((2 * H, 8 * H), jnp.bfloat16),
                    pltpu.VMEM((T, BB, 8 * H), jnp.float32),
                    pltpu.VMEM((T + 2, BB, 2 * H), jnp.bfloat16),
                ],
            ),
            compiler_params=pltpu.CompilerParams(
                dimension_semantics=("arbitrary",),
            ),
        )(emb, w_ih_f, w_ih_b, w_hh_f, w_hh_b,
          (b_ih_f + b_hh_f)[None, :], (b_ih_b + b_hh_b)[None, :],
          cwt, convb, fc_w, fc_b[None, :])

    out = jnp.concatenate(
        [half(x_tokens[i * BB:(i + 1) * BB]) for i in range(Bp // BB)], axis=0)
    return out[:B, :NC]


# speedup vs baseline: 1.0466x; 1.0466x over previous
import jax
import jax.numpy as jnp
from jax import lax
from jax.experimental import pallas as pl
from jax.experimental.pallas import tpu as pltpu

_ORDER = (0, 1, 3, 2)


def _fused_kernel(
    emb_ref,
    wif_ref,
    wib_ref,
    whf_ref,
    whb_ref,
    bf_ref,
    bb_ref,
    cwt_ref,
    convb_ref,
    fcw_ref,
    fcb_ref,
    out_ref,
    wih_s,
    whh_s,
    xg_ref,
    hall_ref,
):
    T, BB, E = emb_ref.shape
    H4, H = whf_ref.shape
    H2 = 2 * H
    G8 = 8 * H
    Cp = convb_ref.shape[1]
    CP3 = 3 * Cp
    NC = fcw_ref.shape[0]
    NCp = out_ref.shape[1]
    TC = 8 if T % 8 == 0 else 1

    def pack_T(wf_ref2, wb_ref2, ncols):
        z = jnp.zeros((H, ncols), jnp.float32)
        blocks = []
        for g in _ORDER:
            blocks.append(jnp.concatenate(
                [wf_ref2[g * H:(g + 1) * H, :], z], axis=1))
            blocks.append(jnp.concatenate(
                [z, wb_ref2[g * H:(g + 1) * H, :]], axis=1))
        return jnp.concatenate(blocks, axis=0)

    wih_s[...] = jnp.transpose(pack_T(wif_ref, wib_ref, E)).astype(jnp.bfloat16)
    whh_s[...] = jnp.transpose(pack_T(whf_ref, whb_ref, H)).astype(jnp.bfloat16)
    bias = jnp.concatenate(
        [jnp.concatenate([bf_ref[:, g * H:(g + 1) * H],
                          bb_ref[:, g * H:(g + 1) * H]], axis=1)
         for g in _ORDER], axis=1)

    wih = wih_s[...]
    whh = whh_s[...]
    hall_ref[0] = jnp.zeros((BB, H2), jnp.bfloat16)
    hall_ref[T + 1] = jnp.zeros((BB, H2), jnp.bfloat16)

    for c in range(T // TC):
        fwd = emb_ref[pl.ds(c * TC, TC)]
        bwd = jnp.stack(
            [emb_ref[T - 1 - (c * TC + k)] for k in range(TC)], axis=0)
        comb = jnp.concatenate([fwd, bwd], axis=-1).astype(
            jnp.bfloat16).reshape(TC * BB, 2 * E)
        xg = jnp.dot(comb, wih, preferred_element_type=jnp.float32) + bias
        xg_ref[pl.ds(c * TC, TC)] = xg.reshape(TC, BB, G8)

    BH = BB // 2
    h0 = jnp.zeros((BH, H2), jnp.bfloat16)
    c0 = jnp.zeros((BH, H2), jnp.float32)

    def substep(t, h, cc, lo):
        gates = (jnp.dot(h, whh, preferred_element_type=jnp.float32)
                 + xg_ref[t, pl.ds(lo, BH)])
        s = 0.5 * jnp.tanh(0.5 * gates[:, :3 * H2]) + 0.5
        g = jnp.tanh(gates[:, 3 * H2:])
        c_new = s[:, H2:2 * H2] * cc + s[:, :H2] * g
        h_new = s[:, 2 * H2:3 * H2] * jnp.tanh(c_new)
        hb = h_new.astype(jnp.bfloat16)
        hall_ref[t + 1, pl.ds(lo, BH), :H] = hb[:, :H]
        hall_ref[T - t, pl.ds(lo, BH), H:] = hb[:, H:]
        return hb, c_new

    def step(t, carry):
        hA, cA, hB, cB = carry
        hA, cA = substep(t, hA, cA, 0)
        hB, cB = substep(t, hB, cB, BH)
        return hA, cA, hB, cB

    lax.fori_loop(0, T, step, (h0, c0, h0, c0), unroll=8)

    def tap(k, rows):
        return cwt_ref[rows, pl.ds(k * Cp, Cp)]
    w1 = jnp.concatenate(
        [tap(0, slice(0, H)), tap(0, slice(H, 2 * H)),
         tap(1, slice(0, H)), tap(1, slice(H, 2 * H)),
         tap(2, slice(0, H)), tap(2, slice(H, 2 * H))], axis=0)
    zc = jnp.zeros((H2, Cp), jnp.float32)
    wacc2 = jnp.concatenate(
        [jnp.concatenate([w1, zc], axis=0),
         jnp.concatenate([zc, w1], axis=0)], axis=1).astype(jnp.bfloat16)

    m2 = jnp.full((BB, 2 * Cp), -jnp.inf, dtype=jnp.float32)
    for p in range(T // 2):
        win = jnp.concatenate(
            [hall_ref[2 * p], hall_ref[2 * p + 1],
             hall_ref[2 * p + 2], hall_ref[2 * p + 3]], axis=1)
        m2 = jnp.maximum(m2, jnp.dot(win, wacc2,
                                     preferred_element_type=jnp.float32))
    m = jnp.maximum(m2[:, :Cp], m2[:, Cp:])
    if T % 2:
        t = T - 1
        win = jnp.concatenate(
            [hall_ref[t], hall_ref[t + 1], hall_ref[t + 2]], axis=1)
        m = jnp.maximum(m, jnp.dot(
            win, wacc2[:3 * H2, :Cp], preferred_element_type=jnp.float32))
    pooled = jnp.maximum(m + convb_ref[...], 0.0)

    logits = lax.dot_general(
        pooled.astype(jnp.bfloat16), fcw_ref[...].astype(jnp.bfloat16),
        (((1,), (1,)), ((), ())),
        preferred_element_type=jnp.float32) + fcb_ref[...]
    out_ref[...] = jnp.concatenate(
        [logits, jnp.zeros((BB, NCp - NC), jnp.float32)], axis=1)


def kernel(x_tokens, embedding, w_ih_f, w_hh_f, b_ih_f, b_hh_f,
           w_ih_b, w_hh_b, b_ih_b, b_hh_b, conv_w, conv_b, fc_w, fc_b):
    B, T = x_tokens.shape
    E = embedding.shape[1]
    H = w_hh_f.shape[1]
    C = conv_w.shape[0]
    NC = fc_b.shape[0]

    BB = 512
    Bp = ((B + BB - 1) // BB) * BB
    Cp = 128
    NCp = 128

    cwt = jnp.transpose(conv_w, (1, 2, 0))
    cwt = jnp.pad(cwt, ((0, 0), (0, 0), (0, Cp - C))).reshape(2 * H, 3 * Cp)
    convb = jnp.zeros((1, Cp), jnp.float32).at[0, :C].set(conv_b)

    if Bp != B:
        x_tokens = jnp.pad(x_tokens, ((0, Bp - B), (0, 0)))

    def half(tok_blk):
        emb = embedding[tok_blk.T]
        return pl.pallas_call(
            _fused_kernel,
            out_shape=jax.ShapeDtypeStruct((BB, NCp), jnp.float32),
            grid_spec=pltpu.PrefetchScalarGridSpec(
                num_scalar_prefetch=0,
                grid=(1,),
                in_specs=[
                    pl.BlockSpec((T, BB, E), lambda i: (0, 0, 0)),
                    pl.BlockSpec((4 * H, E), lambda i: (0, 0)),
                    pl.BlockSpec((4 * H, E), lambda i: (0, 0)),
                    pl.BlockSpec((4 * H, H), lambda i: (0, 0)),
                    pl.BlockSpec((4 * H, H), lambda i: (0, 0)),
                    pl.BlockSpec((1, 4 * H), lambda i: (0, 0)),
                    pl.BlockSpec((1, 4 * H), lambda i: (0, 0)),
                    pl.BlockSpec((2 * H, 3 * Cp), lambda i: (0, 0)),
                    pl.BlockSpec((1, Cp), lambda i: (0, 0)),
                    pl.BlockSpec((NC, C), lambda i: (0, 0)),
                    pl.BlockSpec((1, NC), lambda i: (0, 0)),
                ],
                out_specs=pl.BlockSpec((BB, NCp), lambda i: (0, 0)),
                scratch_shapes=[
                    pltpu.VMEM((2 * E, 8 * H), jnp.bfloat16),
                    pltpu.VMEM((2 * H, 8 * H), jnp.bfloat16),
                    pltpu.VMEM((T, BB, 8 * H), jnp.float32),
                    pltpu.VMEM((T + 2, BB, 2 * H), jnp.bfloat16),
                ],
            ),
            compiler_params=pltpu.CompilerParams(
                dimension_semantics=("arbitrary",),
            ),
        )(emb, w_ih_f, w_ih_b, w_hh_f, w_hh_b,
          (b_ih_f + b_hh_f)[None, :], (b_ih_b + b_hh_b)[None, :],
          cwt, convb, fc_w, fc_b[None, :])

    out = jnp.concatenate(
        [half(x_tokens[i * BB:(i + 1) * BB]) for i in range(Bp // BB)], axis=0)
    return out[:B, :NC]
